# baseline (speedup 1.0000x reference)
import functools

import jax
import jax.numpy as jnp
from jax.experimental import pallas as pl
from jax.experimental.pallas import tpu as pltpu

_D = 128
_R = 5
_E = 60000
_N = 25000
_EDEC = 100000

_PREC = jax.lax.Precision.HIGHEST
_SQRT_HALF = 0.7071067811865476


def _gelu_exact(x):
    return 0.5 * x * (1.0 + jax.lax.erf(x * _SQRT_HALF))


def _stage_a_body(rf_ref, pw_ref, rsw_ref, rw_ref, pa_ref, ra_ref, rfm_ref):
    rf = rf_ref[0]
    pw = pw_ref[0, 0]
    rsw = rsw_ref[0, 0]
    rw = rw_ref[0, 0]
    gates = jnp.concatenate([pw, rsw], axis=1)
    g = jax.nn.sigmoid(jnp.dot(rf, gates, precision=_PREC))
    pa_ref[0, 0] = g[:, 0:1]
    ra_ref[0, 0] = g[:, 1:2]
    rfm_ref[0, 0] = jnp.dot(rf, rw, precision=_PREC)


def _stage_a(review_feat, pw_all, rsw_all, rw_all, be=6000):
    nbe = _E // be
    grid = (2 * _R, nbe)

    def rf_map(g, e):
        return (g % _R, e, 0)

    def w_map(g, e):
        return (g // _R, g % _R, 0, 0)

    def o2_map(g, e):
        return (g // _R, g % _R, e, 0)

    def o3_map(g, e):
        return (g // _R, g % _R, e, 0)

    return pl.pallas_call(
        _stage_a_body,
        grid=grid,
        in_specs=[
            pl.BlockSpec((1, be, _D), rf_map),
            pl.BlockSpec((1, 1, _D, 1), w_map),
            pl.BlockSpec((1, 1, _D, 1), w_map),
            pl.BlockSpec((1, 1, _D, _D), w_map),
        ],
        out_specs=[
            pl.BlockSpec((1, 1, be, 1), o2_map),
            pl.BlockSpec((1, 1, be, 1), o2_map),
            pl.BlockSpec((1, 1, be, _D), o3_map),
        ],
        out_shape=[
            jax.ShapeDtypeStruct((2, _R, _E, 1), jnp.float32),
            jax.ShapeDtypeStruct((2, _R, _E, 1), jnp.float32),
            jax.ShapeDtypeStruct((2, _R, _E, _D), jnp.float32),
        ],
    )(review_feat, pw_all, rsw_all, rw_all)


def _stage_b_body(pre_ref, ci_ref, fcw_ref, fcb_ref, w1_ref, out_ref, dec_ref):
    pre = pre_ref[0]
    ci = ci_ref[0]
    h = _gelu_exact(pre * ci)
    o = jnp.dot(h, fcw_ref[0], precision=_PREC) + fcb_ref[0][0]
    out_ref[0] = o
    dec_ref[0] = jnp.dot(o, w1_ref[0], precision=_PREC)


def _stage_b(pre2, ci2, fcw2, fcb2, w12, bn=5000):
    grid = (2, _N // bn)
    return pl.pallas_call(
        _stage_b_body,
        grid=grid,
        in_specs=[
            pl.BlockSpec((1, bn, _D), lambda d, i: (d, i, 0)),
            pl.BlockSpec((1, bn, 1), lambda d, i: (d, i, 0)),
            pl.BlockSpec((1, _D, _D), lambda d, i: (d, 0, 0)),
            pl.BlockSpec((1, 1, _D), lambda d, i: (d, 0, 0)),
            pl.BlockSpec((1, _D, _D), lambda d, i: (d, 0, 0)),
        ],
        out_specs=[
            pl.BlockSpec((1, bn, _D), lambda d, i: (d, i, 0)),
            pl.BlockSpec((1, bn, _D), lambda d, i: (d, i, 0)),
        ],
        out_shape=[
            jax.ShapeDtypeStruct((2, _N, _D), jnp.float32),
            jax.ShapeDtypeStruct((2, _N, _D), jnp.float32),
        ],
    )(pre2, ci2, fcw2, fcb2, w12)


def _stage_c_body(h_ref, w2_ref, pw_ref, out_ref):
    g = _gelu_exact(h_ref[...])
    t = jnp.dot(g, w2_ref[...], precision=_PREC)
    out_ref[...] = jnp.dot(t, pw_ref[...], precision=_PREC)


def _stage_c(husum, dec_lin2, pred_w, be=10000):
    grid = (_EDEC // be,)
    return pl.pallas_call(
        _stage_c_body,
        grid=grid,
        in_specs=[
            pl.BlockSpec((be, _D), lambda i: (i, 0)),
            pl.BlockSpec((_D, _D), lambda i: (0, 0)),
            pl.BlockSpec((_D, _R), lambda i: (0, 0)),
        ],
        out_specs=pl.BlockSpec((be, _R), lambda i: (i, 0)),
        out_shape=jax.ShapeDtypeStruct((_EDEC, _R), jnp.float32),
    )(husum, dec_lin2, pred_w)


def kernel(ufeat, ifeat, edge_index, review_feat, cj_user, ci_user, cj_movie, ci_movie, dec_edge_index, W_u2m, W_m2u, prob_w_u2m, rev_score_w_u2m, rev_w_u2m, prob_w_m2u, rev_score_w_m2u, rev_w_m2u, ufc_w, ufc_b, ifc_w, ifc_b, dec_lin1, dec_lin2, pred_w):
    pw_all = jnp.stack([prob_w_u2m, prob_w_m2u])
    rsw_all = jnp.stack([rev_score_w_u2m, rev_score_w_m2u])
    rw_all = jnp.stack([rev_w_u2m, rev_w_m2u])

    pa, ra, rfm = _stage_a(review_feat, pw_all, rsw_all, rw_all)
    pa, ra = pa[..., 0], ra[..., 0]

    src_u = edge_index[:, 0, :]
    dst_m = edge_index[:, 1, :]

    def one_dir(d, W_all, src, dst, cj):
        wg = jax.vmap(lambda W, i: W[i])(W_all, src)
        cjs = cj[:, 0][src]
        a = (pa[d] * cjs)[..., None]
        b = (ra[d] * cjs)[..., None]
        m = wg * a + rfm[d] * b
        return jax.ops.segment_sum(
            m.reshape(_R * _E, _D), dst.reshape(-1), num_segments=_N)

    movie_pre = one_dir(0, W_u2m, src_u, dst_m, cj_user)
    user_pre = one_dir(1, W_m2u, dst_m, src_u, cj_movie)

    pre2 = jnp.stack([user_pre, movie_pre])
    ci2 = jnp.stack([ci_user, ci_movie])
    fcw2 = jnp.stack([ufc_w, ifc_w])
    fcb2 = jnp.stack([ufc_b, ifc_b])[:, None, :]
    w12 = jnp.stack([dec_lin1[:_D], dec_lin1[_D:]])

    out2, dec2 = _stage_b(pre2, ci2, fcw2, fcb2, w12)
    uo, io = out2[0], out2[1]

    husum = dec2[0][dec_edge_index[0]] + dec2[1][dec_edge_index[1]]
    scores = _stage_c(husum, dec_lin2, pred_w)
    return (scores, uo, io)


# speedup vs baseline: 2.9289x; 2.9289x over previous
import functools

import jax
import jax.numpy as jnp
from jax import lax
from jax.experimental import pallas as pl
from jax.experimental.pallas import tpu as pltpu
from jax.experimental.pallas import tpu_sc as plsc

_D = 128
_R = 5
_E = 60000
_N = 25000
_EDEC = 100000

_PREC = jax.lax.Precision.HIGHEST
_SQRT_HALF = 0.7071067811865476

_NC = 2
_NS = 16
_L = 16
_CH = 96
_NCHUNK = _E // _CH
_SPLIT = 12504
_ACC_ROWS = 12800
_ZROWS = _ACC_ROWS // _NS
_FLUSH_CH = 176


def _gelu_exact(x):
    return 0.5 * x * (1.0 + jax.lax.erf(x * _SQRT_HALF))


def _stage_a_body(rf_ref, pw_ref, rsw_ref, rw_ref, pa_ref, rfmb_ref):
    rf = rf_ref[0]
    gates = jnp.concatenate([pw_ref[0, 0], rsw_ref[0, 0]], axis=1)
    g = jax.nn.sigmoid(jnp.dot(rf, gates, precision=_PREC))
    pa_ref[0, 0] = g[:, 0:1]
    rfmb_ref[0, 0] = g[:, 1:2] * jnp.dot(rf, rw_ref[0, 0], precision=_PREC)


def _stage_a(review_feat, pw_all, rsw_all, rw_all, be=6000):
    nbe = _E // be
    grid = (2 * _R, nbe)

    def rf_map(g, e):
        return (g % _R, e, 0)

    def w_map(g, e):
        return (g // _R, g % _R, 0, 0)

    def o2_map(g, e):
        return (g // _R, g % _R, e, 0)

    return pl.pallas_call(
        _stage_a_body,
        grid=grid,
        in_specs=[
            pl.BlockSpec((1, be, _D), rf_map),
            pl.BlockSpec((1, 1, _D, 1), w_map),
            pl.BlockSpec((1, 1, _D, 1), w_map),
            pl.BlockSpec((1, 1, _D, _D), w_map),
        ],
        out_specs=[
            pl.BlockSpec((1, 1, be, 1), o2_map),
            pl.BlockSpec((1, 1, be, _D), o2_map),
        ],
        out_shape=[
            jax.ShapeDtypeStruct((2, _R, _E, 1), jnp.float32),
            jax.ShapeDtypeStruct((2, _R, _E, _D), jnp.float32),
        ],
    )(review_feat, pw_all, rsw_all, rw_all)


def _sc_body(wu_hbm, wm_hbm, eidx_hbm, pa_hbm, rfmb_hbm, cju_hbm, cjm_hbm,
             movie_out, user_out,
             acc, wbuf, rbuf, sidx, sadj, didx, dloc, pav, cjv,
             wsem, rsem, csem):
    cid = lax.axis_index("c")
    tid = lax.axis_index("s")
    lo = cid * _SPLIT
    cnt = _SPLIT - cid * 8

    for di, (w_hbm, cj_hbm, s_src, s_dst, out_hbm) in enumerate((
            (wu_hbm, cju_hbm, 0, 1, movie_out),
            (wm_hbm, cjm_hbm, 1, 0, user_out))):
        def zrow(i, _):
            for k in range(8):
                wbuf[i, pl.ds(k * _L, _L)] = jnp.zeros((_L,), jnp.float32)
            return ()
        lax.fori_loop(0, _CH, zrow, ())
        zbase = tid * _ZROWS
        nfull = _ZROWS // _CH
        for j in range(nfull):
            pltpu.sync_copy(wbuf.at[pl.ds(0, _CH)],
                            acc.at[pl.ds(zbase + j * _CH, _CH)])
        rem = _ZROWS - nfull * _CH
        if rem:
            pltpu.sync_copy(wbuf.at[pl.ds(0, rem)],
                            acc.at[pl.ds(zbase + nfull * _CH, rem)])
        plsc.subcore_barrier()

        def rating_body(r, _):
            def chunk_body(c, _):
                e0 = pl.multiple_of(c * _CH, 8)
                pltpu.sync_copy(
                    eidx_hbm.at[pl.ds((r * 2 + s_src) * _E + e0, _CH)], sidx)
                pltpu.sync_copy(
                    eidx_hbm.at[pl.ds((r * 2 + s_dst) * _E + e0, _CH)], didx)
                pltpu.sync_copy(
                    pa_hbm.at[pl.ds((di * _R + r) * _E + e0, _CH)], pav)
                for g in range(_CH // _L):
                    sl = pl.ds(g * _L, _L)
                    s16 = sidx[sl]
                    sadj[sl] = s16 + r * _N
                    d16 = didx[sl] - lo
                    ok = (d16 >= 0) & (d16 < cnt)
                    dloc[sl] = jnp.where(
                        ok, d16,
                        _SPLIT + lax.broadcasted_iota(jnp.int32, (_L,), 0))
                cp_c = pltpu.async_copy(cj_hbm.at[sidx], cjv, csem)
                cp_w = pltpu.async_copy(w_hbm.at[sadj], wbuf, wsem)
                cp_r = pltpu.async_copy(
                    rfmb_hbm.at[di, r, pl.ds(e0, _CH)], rbuf, rsem)
                cp_c.wait()
                cp_w.wait()
                cp_r.wait()

                def group_body(g, _):
                    sl = pl.ds(g * _L, _L)
                    cj16 = cjv[sl]
                    a16 = pav[sl] * cj16
                    for j in range(_L):
                        a_s = jnp.full((_L,), a16[j], jnp.float32)
                        c_s = jnp.full((_L,), cj16[j], jnp.float32)
                        e = g * _L + j
                        for k in range(8):
                            slk = pl.ds(k * _L, _L)
                            wbuf[e, slk] = (a_s * wbuf[e, slk]
                                            + c_s * rbuf[e, slk])
                    return ()
                lax.fori_loop(0, _CH // _L, group_body, ())

                pltpu.sync_copy(wbuf, acc.at[dloc], add=True)
                return ()

            nmine = (_NCHUNK - tid + _NS - 1) // _NS
            def tile_chunk(i, _):
                return chunk_body(tid + i * _NS, ())
            lax.fori_loop(0, nmine, tile_chunk, ())
            return ()
        lax.fori_loop(0, _R, rating_body, ())

        plsc.subcore_barrier()
        nfl = 12496 // _FLUSH_CH
        def flush_body(i, _):
            c = tid + i * _NS
            pltpu.sync_copy(
                acc.at[pl.ds(c * _FLUSH_CH, _FLUSH_CH)],
                out_hbm.at[pl.ds(lo + c * _FLUSH_CH, _FLUSH_CH)])
            return ()
        nflm = (nfl - tid + _NS - 1) // _NS
        lax.fori_loop(0, nflm, flush_body, ())
        @pl.when((cid == 0) & (tid == 0))
        def _():
            pltpu.sync_copy(acc.at[pl.ds(12496, 8)],
                            out_hbm.at[pl.ds(12496, 8)])
        plsc.subcore_barrier()


def _sc_message_passing(W_u2m, W_m2u, edge_index, pa, rfmb, cju, cjm):
    mesh = plsc.VectorSubcoreMesh(
        core_axis_name="c", subcore_axis_name="s",
        num_cores=_NC, num_subcores=_NS)
    f = pl.kernel(
        _sc_body,
        out_type=[
            jax.ShapeDtypeStruct((_N, _D), jnp.float32),
            jax.ShapeDtypeStruct((_N, _D), jnp.float32),
        ],
        mesh=mesh,
        scratch_types=[
            pltpu.VMEM_SHARED((_ACC_ROWS, _D), jnp.float32),
            pltpu.VMEM((_CH, _D), jnp.float32),
            pltpu.VMEM((_CH, _D), jnp.float32),
            pltpu.VMEM((_CH,), jnp.int32),
            pltpu.VMEM((_CH,), jnp.int32),
            pltpu.VMEM((_CH,), jnp.int32),
            pltpu.VMEM((_CH,), jnp.int32),
            pltpu.VMEM((_CH,), jnp.float32),
            pltpu.VMEM((_CH,), jnp.float32),
            pltpu.SemaphoreType.DMA,
            pltpu.SemaphoreType.DMA,
            pltpu.SemaphoreType.DMA,
        ],
    )
    return f(W_u2m.reshape(_R * _N, _D), W_m2u.reshape(_R * _N, _D),
             edge_index.reshape(-1), pa.reshape(-1), rfmb, cju, cjm)


def _stage_b_body(pre_ref, ci_ref, fcw_ref, fcb_ref, w1_ref, out_ref, dec_ref):
    h = _gelu_exact(pre_ref[0] * ci_ref[0])
    o = jnp.dot(h, fcw_ref[0], precision=_PREC) + fcb_ref[0][0]
    out_ref[0] = o
    dec_ref[0] = jnp.dot(o, w1_ref[0], precision=_PREC)


def _stage_b(pre2, ci2, fcw2, fcb2, w12, bn=5000):
    grid = (2, _N // bn)
    return pl.pallas_call(
        _stage_b_body,
        grid=grid,
        in_specs=[
            pl.BlockSpec((1, bn, _D), lambda d, i: (d, i, 0)),
            pl.BlockSpec((1, bn, 1), lambda d, i: (d, i, 0)),
            pl.BlockSpec((1, _D, _D), lambda d, i: (d, 0, 0)),
            pl.BlockSpec((1, 1, _D), lambda d, i: (d, 0, 0)),
            pl.BlockSpec((1, _D, _D), lambda d, i: (d, 0, 0)),
        ],
        out_specs=[
            pl.BlockSpec((1, bn, _D), lambda d, i: (d, i, 0)),
            pl.BlockSpec((1, bn, _D), lambda d, i: (d, i, 0)),
        ],
        out_shape=[
            jax.ShapeDtypeStruct((2, _N, _D), jnp.float32),
            jax.ShapeDtypeStruct((2, _N, _D), jnp.float32),
        ],
    )(pre2, ci2, fcw2, fcb2, w12)


def _stage_c_body(h_ref, w2_ref, pw_ref, out_ref):
    g = _gelu_exact(h_ref[...])
    t = jnp.dot(g, w2_ref[...], precision=_PREC)
    out_ref[...] = jnp.dot(t, pw_ref[...], precision=_PREC)


def _stage_c(husum, dec_lin2, pred_w, be=10000):
    grid = (_EDEC // be,)
    return pl.pallas_call(
        _stage_c_body,
        grid=grid,
        in_specs=[
            pl.BlockSpec((be, _D), lambda i: (i, 0)),
            pl.BlockSpec((_D, _D), lambda i: (0, 0)),
            pl.BlockSpec((_D, _R), lambda i: (0, 0)),
        ],
        out_specs=pl.BlockSpec((be, _R), lambda i: (i, 0)),
        out_shape=jax.ShapeDtypeStruct((_EDEC, _R), jnp.float32),
    )(husum, dec_lin2, pred_w)


def kernel(ufeat, ifeat, edge_index, review_feat, cj_user, ci_user, cj_movie, ci_movie, dec_edge_index, W_u2m, W_m2u, prob_w_u2m, rev_score_w_u2m, rev_w_u2m, prob_w_m2u, rev_score_w_m2u, rev_w_m2u, ufc_w, ufc_b, ifc_w, ifc_b, dec_lin1, dec_lin2, pred_w):
    pw_all = jnp.stack([prob_w_u2m, prob_w_m2u])
    rsw_all = jnp.stack([rev_score_w_u2m, rev_score_w_m2u])
    rw_all = jnp.stack([rev_w_u2m, rev_w_m2u])

    pa4, rfmb = _stage_a(review_feat, pw_all, rsw_all, rw_all)
    pa = pa4[..., 0]

    movie_pre, user_pre = _sc_message_passing(
        W_u2m, W_m2u, edge_index.astype(jnp.int32), pa, rfmb,
        cj_user.reshape(_N), cj_movie.reshape(_N))

    pre2 = jnp.stack([user_pre, movie_pre])
    ci2 = jnp.stack([ci_user, ci_movie])
    fcw2 = jnp.stack([ufc_w, ifc_w])
    fcb2 = jnp.stack([ufc_b, ifc_b])[:, None, :]
    w12 = jnp.stack([dec_lin1[:_D], dec_lin1[_D:]])

    out2, dec2 = _stage_b(pre2, ci2, fcw2, fcb2, w12)
    uo, io = out2[0], out2[1]

    husum = dec2[0][dec_edge_index[0]] + dec2[1][dec_edge_index[1]]
    scores = _stage_c(husum, dec_lin2, pred_w)
    return (scores, uo, io)
